# baseline (speedup 1.0000x reference)
import jax
import jax.numpy as jnp
from jax.experimental import pallas as pl
from jax.experimental.pallas import tpu as pltpu

DEP = 128
SLOT = 128
GROUPS = 2
GROUP_SIZE = 1024
IB = 128


def _body(addr_ref, embt_ref, freq_ref, mem_ref, out_ref):
    ib = pl.program_id(1)
    a = addr_ref[...]
    ft = embt_ref[...] * freq_ref[...]
    contrib = jnp.sum(a * ft[:, :, None], axis=1)

    @pl.when(ib == 0)
    def _init():
        out_ref[...] = mem_ref[...] + contrib[None]

    @pl.when(ib != 0)
    def _acc():
        out_ref[...] += contrib[None]


def kernel(batch_address, batch_embedding, batch_frequency, memory_matrix):
    embt = batch_embedding.T
    freq = batch_frequency[None, :]
    n_ib = GROUP_SIZE // IB
    grid = (GROUPS, n_ib)
    return pl.pallas_call(
        _body,
        grid=grid,
        in_specs=[
            pl.BlockSpec((DEP, IB, SLOT), lambda g, ib: (0, g * n_ib + ib, 0)),
            pl.BlockSpec((DEP, IB), lambda g, ib: (0, g * n_ib + ib)),
            pl.BlockSpec((1, IB), lambda g, ib: (0, g * n_ib + ib)),
            pl.BlockSpec((1, DEP, SLOT), lambda g, ib: (g, 0, 0)),
        ],
        out_specs=pl.BlockSpec((1, DEP, SLOT), lambda g, ib: (g, 0, 0)),
        out_shape=jax.ShapeDtypeStruct((GROUPS, DEP, SLOT), jnp.float32),
    )(batch_address, embt, freq, memory_matrix)


# speedup vs baseline: 1.0641x; 1.0641x over previous
import jax
import jax.numpy as jnp
from jax.experimental import pallas as pl
from jax.experimental.pallas import tpu as pltpu

DEP = 128
SLOT = 128
GROUPS = 2
GROUP_SIZE = 1024
IB = 256


def _body(addr_ref, embt_ref, freq_ref, mem_ref, out_ref):
    ib = pl.program_id(1)
    a = addr_ref[...]
    ft = embt_ref[...] * freq_ref[...]
    contrib = jnp.sum(a * ft[:, :, None], axis=1)

    @pl.when(ib == 0)
    def _init():
        out_ref[...] = mem_ref[...] + contrib[None]

    @pl.when(ib != 0)
    def _acc():
        out_ref[...] += contrib[None]


def kernel(batch_address, batch_embedding, batch_frequency, memory_matrix):
    embt = batch_embedding.T
    freq = batch_frequency[None, :]
    n_ib = GROUP_SIZE // IB
    grid = (GROUPS, n_ib)
    return pl.pallas_call(
        _body,
        grid=grid,
        in_specs=[
            pl.BlockSpec((DEP, IB, SLOT), lambda g, ib: (0, g * n_ib + ib, 0)),
            pl.BlockSpec((DEP, IB), lambda g, ib: (0, g * n_ib + ib)),
            pl.BlockSpec((1, IB), lambda g, ib: (0, g * n_ib + ib)),
            pl.BlockSpec((1, DEP, SLOT), lambda g, ib: (g, 0, 0)),
        ],
        out_specs=pl.BlockSpec((1, DEP, SLOT), lambda g, ib: (g, 0, 0)),
        out_shape=jax.ShapeDtypeStruct((GROUPS, DEP, SLOT), jnp.float32),
    )(batch_address, embt, freq, memory_matrix)
